# Initial kernel scaffold; baseline (speedup 1.0000x reference)
import jax
import jax.numpy as jnp
from jax.experimental import pallas as pl


def kernel(x, table):
    raise NotImplementedError("write your pallas kernel here")


# speedup vs baseline: 1.2852x; 1.2852x over previous
import functools

import jax
import jax.numpy as jnp
from jax import lax
from jax.experimental import pallas as pl
from jax.experimental.pallas import tpu as pltpu
from jax.experimental.pallas import tpu_sc as plsc

_PAD_IDX = 300
_V = 301
_D = 64
_B = 4096 * 50 * 2

_info = plsc.get_sparse_core_info()
_NC, _NS, _L = _info.num_cores, _info.num_subcores, _info.num_lanes
_NW = _NC * _NS
_PER_W = _B // _NW
_CHUNK = 640
_NCHUNK = _PER_W // _CHUNK
_GROUPS = _CHUNK // _L
_UNROLL = 8


@functools.partial(
    pl.kernel,
    mesh=plsc.VectorSubcoreMesh(core_axis_name="c", subcore_axis_name="s"),
    compiler_params=pltpu.CompilerParams(needs_layout_passes=False),
    out_type=jax.ShapeDtypeStruct((_B * _D,), jnp.float32),
    scratch_types=[
        pltpu.VMEM((_V * _D,), jnp.float32),
        pltpu.VMEM((_PER_W,), jnp.int32),
        pltpu.VMEM((_CHUNK * _D,), jnp.float32),
    ],
)
def _embed_sc(x_hbm, table_hbm, out_hbm, table_v, idx_v, out_v):
    wid = lax.axis_index("s") * _NC + lax.axis_index("c")
    base = wid * _PER_W
    pltpu.sync_copy(table_hbm, table_v)
    pltpu.sync_copy(x_hbm.at[pl.ds(base, _PER_W)], idx_v)

    iota64 = lax.iota(jnp.int32, _L) * _D

    def chunk_body(c, carry):
        def grp(g, carry2):
            idxv = idx_v[pl.ds(c * _CHUNK + g * _L, _L)]
            src0 = idxv * _D
            dst0 = iota64 + g * (_L * _D)

            def col_step(t, sd):
                s, d = sd
                for u in range(_UNROLL):
                    vals = plsc.load_gather(table_v, [s + u])
                    plsc.store_scatter(out_v, [d + u], vals)
                return (s + _UNROLL, d + _UNROLL)

            lax.fori_loop(0, _D // _UNROLL, col_step, (src0, dst0),
                          unroll=False)
            return carry2

        lax.fori_loop(0, _GROUPS, grp, 0, unroll=False)
        pltpu.sync_copy(
            out_v,
            out_hbm.at[pl.ds((base + c * _CHUNK) * _D, _CHUNK * _D)],
        )
        return carry

    lax.fori_loop(0, _NCHUNK, chunk_body, 0, unroll=False)


def kernel(x, table):
    idx = x.reshape(-1).astype(jnp.int32)
    tbl = table.at[_PAD_IDX].set(0.0).reshape(-1)
    out = _embed_sc(idx, tbl)
    return out.reshape(x.shape[0], x.shape[1], x.shape[2] * _D)


# speedup vs baseline: 3.8852x; 3.0231x over previous
import functools

import jax
import jax.numpy as jnp
from jax import lax
from jax.experimental import pallas as pl
from jax.experimental.pallas import tpu as pltpu
from jax.experimental.pallas import tpu_sc as plsc

_PAD_IDX = 300
_V = 301
_D = 64
_B = 4096 * 50 * 2

_info = plsc.get_sparse_core_info()
_NC, _NS, _L = _info.num_cores, _info.num_subcores, _info.num_lanes
_NW = _NC * _NS
_PER_W = _B // _NW
_CHUNK = 640
_NCHUNK = _PER_W // _CHUNK
_RU = 4


@functools.partial(
    pl.kernel,
    mesh=plsc.VectorSubcoreMesh(core_axis_name="c", subcore_axis_name="s"),
    compiler_params=pltpu.CompilerParams(needs_layout_passes=False),
    out_type=jax.ShapeDtypeStruct((_B * _D,), jnp.float32),
    scratch_types=[
        pltpu.VMEM((_V * _D,), jnp.float32),
        pltpu.VMEM((_PER_W,), jnp.int32),
        pltpu.VMEM((_CHUNK * _D,), jnp.float32),
    ],
)
def _embed_sc(x_hbm, table_hbm, out_hbm, table_v, idx_v, out_v):
    wid = lax.axis_index("s") * _NC + lax.axis_index("c")
    base = wid * _PER_W
    pltpu.sync_copy(table_hbm, table_v)
    pltpu.sync_copy(x_hbm.at[pl.ds(base, _PER_W)], idx_v)

    def chunk_body(c, carry):
        cbase = c * _CHUNK

        def row_grp(t, carry2):
            idxv = idx_v[pl.ds(cbase + t * _L, _L)]
            dstb = t * _L * _D
            nj = _D // _L

            def pair_loads(u):
                s0 = idxv[u] * _D
                s1 = idxv[u + 1] * _D
                return ([table_v[pl.ds(s0 + j * _L, _L)] for j in range(nj)]
                        + [table_v[pl.ds(s1 + j * _L, _L)]
                           for j in range(nj)])

            vals = pair_loads(0)
            for u in range(2, _L, 2):
                s0 = idxv[u] * _D
                s1 = idxv[u + 1] * _D
                srcs = ([s0 + j * _L for j in range(nj)]
                        + [s1 + j * _L for j in range(nj)])
                nxt = []
                for j in range(2 * nj):
                    nv = table_v[pl.ds(srcs[j], _L)]
                    out_v[pl.ds(dstb + (u - 2) * _D + j * _L, _L)] = vals[j]
                    nxt.append(nv)
                vals = nxt
            for j in range(2 * nj):
                out_v[pl.ds(dstb + (_L - 2) * _D + j * _L, _L)] = vals[j]
            return carry2

        lax.fori_loop(0, _CHUNK // _L, row_grp, 0, unroll=False)
        pltpu.sync_copy(
            out_v,
            out_hbm.at[pl.ds((base + c * _CHUNK) * _D, _CHUNK * _D)],
        )
        return carry

    lax.fori_loop(0, _NCHUNK, chunk_body, 0, unroll=False)


def kernel(x, table):
    idx = x.reshape(-1).astype(jnp.int32)
    tbl = table.at[_PAD_IDX].set(0.0).reshape(-1)
    out = _embed_sc(idx, tbl)
    return out.reshape(x.shape[0], x.shape[1], x.shape[2] * _D)


# speedup vs baseline: 4.1414x; 1.0659x over previous
import functools

import jax
import jax.numpy as jnp
from jax import lax
from jax.experimental import pallas as pl
from jax.experimental.pallas import tpu as pltpu
from jax.experimental.pallas import tpu_sc as plsc

_PAD_IDX = 300
_V = 301
_D = 64
_B = 4096 * 50 * 2

_info = plsc.get_sparse_core_info()
_NC, _NS, _L = _info.num_cores, _info.num_subcores, _info.num_lanes
_NW = _NC * _NS
_PER_W = _B // _NW
_CHUNK = 640
_NCHUNK = _PER_W // _CHUNK


@functools.partial(
    pl.kernel,
    mesh=plsc.VectorSubcoreMesh(core_axis_name="c", subcore_axis_name="s"),
    compiler_params=pltpu.CompilerParams(needs_layout_passes=False),
    out_type=jax.ShapeDtypeStruct((_B * _D,), jnp.float32),
    scratch_types=[
        pltpu.VMEM((_V * _D,), jnp.float32),
        pltpu.VMEM((_PER_W,), jnp.int32),
        pltpu.VMEM((_CHUNK * _D,), jnp.float32),
        pltpu.VMEM((_CHUNK * _D,), jnp.float32),
        pltpu.SemaphoreType.DMA,
        pltpu.SemaphoreType.DMA,
    ],
)
def _embed_sc(x_hbm, table_hbm, out_hbm, table_v, idx_v, out_a, out_b,
              sem_a, sem_b):
    wid = lax.axis_index("s") * _NC + lax.axis_index("c")
    base = wid * _PER_W
    pltpu.sync_copy(table_hbm, table_v)
    pltpu.sync_copy(x_hbm.at[pl.ds(base, _PER_W)], idx_v)

    def fill(c, out_v):
        cbase = c * _CHUNK

        def row_grp(t, carry2):
            idxv = idx_v[pl.ds(cbase + t * _L, _L)]
            dstb = t * _L * _D
            nj = _D // _L

            def pair_loads(u):
                s0 = idxv[u] * _D
                s1 = idxv[u + 1] * _D
                return ([table_v[pl.ds(s0 + j * _L, _L)] for j in range(nj)]
                        + [table_v[pl.ds(s1 + j * _L, _L)]
                           for j in range(nj)])

            vals = pair_loads(0)
            for u in range(2, _L, 2):
                s0 = idxv[u] * _D
                s1 = idxv[u + 1] * _D
                srcs = ([s0 + j * _L for j in range(nj)]
                        + [s1 + j * _L for j in range(nj)])
                nxt = []
                for j in range(2 * nj):
                    nv = table_v[pl.ds(srcs[j], _L)]
                    out_v[pl.ds(dstb + (u - 2) * _D + j * _L, _L)] = vals[j]
                    nxt.append(nv)
                vals = nxt
            for j in range(2 * nj):
                out_v[pl.ds(dstb + (_L - 2) * _D + j * _L, _L)] = vals[j]
            return carry2

        lax.fori_loop(0, _CHUNK // _L, row_grp, 0, unroll=False)

    def out_slice(c):
        return out_hbm.at[pl.ds((base + c * _CHUNK) * _D, _CHUNK * _D)]

    fill(0, out_a)
    pltpu.async_copy(out_a, out_slice(0), sem_a)
    fill(1, out_b)
    pltpu.async_copy(out_b, out_slice(1), sem_b)

    def step(st, carry):
        c0 = st * 2
        pltpu.make_async_copy(out_a, out_slice(c0), sem_a).wait()
        fill(c0, out_a)
        pltpu.async_copy(out_a, out_slice(c0), sem_a)
        pltpu.make_async_copy(out_b, out_slice(c0 + 1), sem_b).wait()
        fill(c0 + 1, out_b)
        pltpu.async_copy(out_b, out_slice(c0 + 1), sem_b)
        return carry

    lax.fori_loop(1, _NCHUNK // 2, step, 0, unroll=False)
    pltpu.make_async_copy(out_a, out_slice(0), sem_a).wait()
    pltpu.make_async_copy(out_b, out_slice(1), sem_b).wait()


def kernel(x, table):
    idx = x.reshape(-1).astype(jnp.int32)
    tbl = table.at[_PAD_IDX].set(0.0).reshape(-1)
    out = _embed_sc(idx, tbl)
    return out.reshape(x.shape[0], x.shape[1], x.shape[2] * _D)


# speedup vs baseline: 19.1534x; 4.6249x over previous
import functools

import jax
import jax.numpy as jnp
from jax import lax
from jax.experimental import pallas as pl
from jax.experimental.pallas import tpu as pltpu
from jax.experimental.pallas import tpu_sc as plsc

_PAD_IDX = 300
_V = 301
_D = 64
_NB = 4096
_NL = 50
_NR = _NL * _NB

_info = plsc.get_sparse_core_info()
_NC, _NS, _L = _info.num_cores, _info.num_subcores, _info.num_lanes
_NW = _NC * _NS
_PER_W = _NR // _NW
_CHUNK = 320
_NCHUNK = _PER_W // _CHUNK
_ISTAGE = 3 * 2 * _NB


@functools.partial(
    pl.kernel,
    mesh=plsc.VectorSubcoreMesh(core_axis_name="c", subcore_axis_name="s"),
    compiler_params=pltpu.CompilerParams(needs_layout_passes=False),
    out_type=jax.ShapeDtypeStruct((_NR * 2 * _D,), jnp.float32),
    scratch_types=[
        pltpu.VMEM((_V * _D,), jnp.float32),
        pltpu.VMEM((_ISTAGE,), jnp.int32),
        pltpu.VMEM((_CHUNK * 2 * _D,), jnp.float32),
        pltpu.VMEM((_CHUNK * 2 * _D,), jnp.float32),
        pltpu.SemaphoreType.DMA,
        pltpu.SemaphoreType.DMA,
    ],
)
def _embed_sc(x_hbm, table_hbm, out_hbm, table_v, idx_v, out_a, out_b,
              sem_a, sem_b):
    wid = lax.axis_index("s") * _NC + lax.axis_index("c")
    rbase = wid * _PER_W
    l_start = jnp.minimum(rbase // _NB, _NL - 3)
    pltpu.sync_copy(table_hbm, table_v)
    pltpu.sync_copy(x_hbm.at[pl.ds(l_start * (2 * _NB), _ISTAGE)], idx_v)

    def fill(c, out_v):
        cbase = rbase + c * _CHUNK

        def row_grp(g, carry2):
            r0 = cbase + g * _L
            ll = r0 // _NB
            bb = r0 % _NB
            pos0 = (ll - l_start) * (2 * _NB) + bb
            idx0 = idx_v[pl.ds(pos0, _L)]
            idx1 = idx_v[pl.ds(pos0 + _NB, _L)]
            dstb = g * _L * (2 * _D)
            nj = _D // _L

            def lane_loads(u):
                s0 = idx0[u] * _D
                s1 = idx1[u] * _D
                return ([table_v[pl.ds(s0 + j * _L, _L)] for j in range(nj)]
                        + [table_v[pl.ds(s1 + j * _L, _L)]
                           for j in range(nj)])

            vals = lane_loads(0)
            for u in range(1, _L):
                s0 = idx0[u] * _D
                s1 = idx1[u] * _D
                srcs = ([s0 + j * _L for j in range(nj)]
                        + [s1 + j * _L for j in range(nj)])
                nxt = []
                for j in range(2 * nj):
                    nv = table_v[pl.ds(srcs[j], _L)]
                    out_v[pl.ds(dstb + (u - 1) * (2 * _D) + j * _L, _L)] = (
                        vals[j])
                    nxt.append(nv)
                vals = nxt
            for j in range(2 * nj):
                out_v[pl.ds(dstb + (_L - 1) * (2 * _D) + j * _L, _L)] = vals[j]
            return carry2

        lax.fori_loop(0, _CHUNK // _L, row_grp, 0, unroll=False)

    def out_slice(c):
        return out_hbm.at[
            pl.ds((rbase + c * _CHUNK) * (2 * _D), _CHUNK * 2 * _D)]

    fill(0, out_a)
    pltpu.async_copy(out_a, out_slice(0), sem_a)
    fill(1, out_b)
    pltpu.async_copy(out_b, out_slice(1), sem_b)

    def step(st, carry):
        c0 = st * 2
        pltpu.make_async_copy(out_a, out_slice(c0), sem_a).wait()
        fill(c0, out_a)
        pltpu.async_copy(out_a, out_slice(c0), sem_a)
        pltpu.make_async_copy(out_b, out_slice(c0 + 1), sem_b).wait()
        fill(c0 + 1, out_b)
        pltpu.async_copy(out_b, out_slice(c0 + 1), sem_b)
        return carry

    lax.fori_loop(1, _NCHUNK // 2, step, 0, unroll=False)
    pltpu.make_async_copy(out_a, out_slice(0), sem_a).wait()
    pltpu.make_async_copy(out_b, out_slice(1), sem_b).wait()


def kernel(x, table):
    idx = x.transpose(1, 2, 0).reshape(-1).astype(jnp.int32)
    tbl = table.at[_PAD_IDX].set(0.0).reshape(-1)
    out = _embed_sc(idx, tbl)
    return out.reshape(_NL, _NB, 2 * _D).transpose(1, 0, 2)
